# baseline (speedup 1.0000x reference)
import functools

import jax
import jax.numpy as jnp
from jax import lax
from jax.experimental import pallas as pl
from jax.experimental.pallas import tpu as pltpu
from jax.experimental.pallas import tpu_sc as plsc

N = 10000
E = 200000
HEADS = 4
HDIM = 64
FDIM = HEADS * HDIM

NCORE = 2
NSUB = 16
NTILE = NCORE * NSUB
C = 32
CHUNKS = 200
BLK = 8
NBLK = CHUNKS // BLK
EPT = C * CHUNKS
E_PAD = EPT * NTILE
RPS = 624
RTAIL = N - NSUB * RPS

_f32 = jnp.float32
_i32 = jnp.int32


def _tc_call(body, out_shapes, *args):
    return pl.pallas_call(body, out_shape=out_shapes)(*args)


def _mm_body(x_ref, w_ref, b_ref, o_ref, *, act):
    y = jnp.dot(x_ref[...], w_ref[...], preferred_element_type=_f32)
    y = y + b_ref[...]
    if act:
        y = jnp.maximum(y, 0.0)
    o_ref[...] = y


def _mm(x, w, b, act=False):
    body = functools.partial(_mm_body, act=act)
    return _tc_call(body, jax.ShapeDtypeStruct((x.shape[0], w.shape[1]), _f32),
                    x, w, b.reshape(1, -1))


def _recip_body(d_ref, o_ref):
    o_ref[...] = 1.0 / (d_ref[0] + d_ref[1] + 1e-16)


def _recip(den):
    d = den.reshape(2, (N * 16) // 128, 128)
    r = _tc_call(_recip_body, jax.ShapeDtypeStruct(d.shape[1:], _f32), d)
    return jnp.pad(r.reshape(N, 16), ((0, 0), (0, 112)))


def _comb_g_body(o0_ref, o1_ref, x_ref, w_ref, br_ref, bs_ref, out_ref):
    a = 0.25 * (o0_ref[...] + o1_ref[...]) + br_ref[...]
    y = a + jnp.dot(x_ref[...], w_ref[...], preferred_element_type=_f32) + bs_ref[...]
    out_ref[...] = jnp.maximum(y, 0.0)


def _comb_p_body(g0_ref, g1_ref, p0_ref, p1_ref, x_ref, w_ref,
                 b1_ref, b2_ref, bs_ref, out_ref):
    a = (0.25 * (g0_ref[...] + g1_ref[...]) + b1_ref[...]
         + 0.25 * (p0_ref[...] + p1_ref[...]) + b2_ref[...])
    y = a + jnp.dot(x_ref[...], w_ref[...], preferred_element_type=_f32) + bs_ref[...]
    out_ref[...] = jnp.maximum(y, 0.0)


def _final_body(xg_ref, xp_ref, wg_ref, wp_ref, b_ref, o_ref):
    y = (jnp.dot(xg_ref[...], wg_ref[...], preferred_element_type=_f32)
         + jnp.dot(xp_ref[...], wp_ref[...], preferred_element_type=_f32)
         + b_ref[...])
    o_ref[...] = y


def _sc_mesh():
    return plsc.VectorSubcoreMesh(core_axis_name="c", subcore_axis_name="s")


def _acc_slices(sid, fn):
    fn(sid * RPS, RPS)

    @pl.when(sid == NSUB - 1)
    def _tail():
        fn(NSUB * RPS, RTAIL)


def _zero_fill(zbuf, w):
    zero16 = jnp.zeros((16,), _f32)
    for i in range(16):
        for j in range(w // 16):
            zbuf[i, pl.ds(j * 16, 16)] = zero16


def _zero_acc(sid, zbuf, acc_sh):
    def zslices(lo, sz):
        def b(j, c):
            pltpu.sync_copy(zbuf, acc_sh.at[pl.ds(lo + j * 16, 16)])
            return c
        lax.fori_loop(0, sz // 16, b, 0)
    _acc_slices(sid, zslices)


def _pass1_body(xl0, xr0, xl1, xr1, xl2, xr2,
                s0, d0, s1, d1, s2, d2, a0, a1, a2,
                ex0, ex1, ex2, dn0, dn1, dn2,
                src_blk, dst_blk, xl_rows, xr_rows, att_v, exb, zbuf,
                den_sh, sem1, sem2):
    cid = lax.axis_index("c")
    sid = lax.axis_index("s")
    wid = sid * NCORE + cid
    rb = wid * CHUNKS

    rels = ((xl0, xr0, s0, d0, a0, ex0, dn0),
            (xl1, xr1, s1, d1, a1, ex1, dn1),
            (xl2, xr2, s2, d2, a2, ex2, dn2))

    _zero_fill(zbuf, 16)
    _zero_acc(sid, zbuf, den_sh)

    lane = lax.iota(_i32, 16)
    lanemask = lane < HEADS

    for r, (xl_hbm, xr_hbm, src_hbm, dst_hbm, att_hbm, ex_out, den_out) \
            in enumerate(rels):
        pltpu.sync_copy(att_hbm, att_v)
        plsc.subcore_barrier()

        attv = [att_v[pl.ds(i * 16, 16)] for i in range(16)]

        def blk_body(b, carry):
            pltpu.sync_copy(src_hbm.at[pl.ds(rb + b * BLK, BLK)], src_blk)
            pltpu.sync_copy(dst_hbm.at[pl.ds(rb + b * BLK, BLK)], dst_blk)
            lax.fori_loop(0, BLK, functools.partial(chunk, b), 0)
            return carry

        def chunk(b, j, carry):
            k = b * BLK + j
            cp1 = pltpu.async_copy(xl_hbm.at[src_blk.at[j]], xl_rows, sem1)
            cp2 = pltpu.async_copy(xr_hbm.at[dst_blk.at[j]], xr_rows, sem2)
            cp1.wait()
            cp2.wait()
            base_edge = (rb + k) * C
            for e in range(C):
                row = jnp.zeros((16,), _f32)
                for h in range(HEADS):
                    acc = None
                    for c4 in range(4):
                        off = h * HDIM + c4 * 16
                        s = (xl_rows[e, pl.ds(off, 16)]
                             + xr_rows[e, pl.ds(off, 16)])
                        s = jnp.maximum(s, s * 0.2)
                        t = s * attv[h * 4 + c4]
                        acc = t if acc is None else acc + t
                    row = jnp.where(lane == h, jnp.sum(acc), row)
                ex = jnp.exp(row)
                m = jnp.logical_and(lanemask, base_edge + e < E)
                exb[e, :] = jnp.where(m, ex, 0.0)
            pltpu.sync_copy(exb, ex_out.at[pl.ds(base_edge, C)])
            pltpu.sync_copy(exb, den_sh.at[dst_blk.at[j]], add=True)
            return carry

        lax.fori_loop(0, NBLK, blk_body, 0)

        plsc.subcore_barrier()
        _acc_slices(sid, lambda lo, sz: pltpu.sync_copy(
            den_sh.at[pl.ds(lo, sz)], den_out.at[cid, pl.ds(lo, sz)]))
        if r < 2:
            _zero_acc(sid, zbuf, den_sh)


def _sc_pass1(tables, edges2, atts):
    ex_t = jax.ShapeDtypeStruct((E_PAD, 16), _f32)
    dn_t = jax.ShapeDtypeStruct((NCORE, N, 16), _f32)
    kern = pl.kernel(
        _pass1_body,
        out_type=(ex_t, ex_t, ex_t, dn_t, dn_t, dn_t),
        mesh=_sc_mesh(),
        compiler_params=pltpu.CompilerParams(needs_layout_passes=False),
        scratch_types=[
            pltpu.VMEM((BLK, C), _i32),
            pltpu.VMEM((BLK, C), _i32),
            pltpu.VMEM((C, FDIM), _f32),
            pltpu.VMEM((C, FDIM), _f32),
            pltpu.VMEM((FDIM,), _f32),
            pltpu.VMEM((C, 16), _f32),
            pltpu.VMEM((16, 16), _f32),
            pltpu.VMEM_SHARED((N, 16), _f32),
            pltpu.SemaphoreType.DMA,
            pltpu.SemaphoreType.DMA,
        ],
    )
    (xl0, xr0), (xl1, xr1), (xl2, xr2) = tables
    (s0, d0), (s1, d1), (s2, d2) = edges2
    return kern(xl0, xr0, xl1, xr1, xl2, xr2,
                s0, d0, s1, d1, s2, d2, atts[0], atts[1], atts[2])


def _pass2_body(xl0, xl1, xl2, s0, d0, s1, d1, s2, d2,
                e0, e1, e2, r0, r1, r2,
                o0, o1, o2,
                src_blk, dst_blk, xl_rows, ex_rows, r_rows, vbuf, zbuf,
                out_sh, sem1, sem2):
    cid = lax.axis_index("c")
    sid = lax.axis_index("s")
    wid = sid * NCORE + cid
    rb = wid * CHUNKS

    rels = ((xl0, s0, d0, e0, r0, o0),
            (xl1, s1, d1, e1, r1, o1),
            (xl2, s2, d2, e2, r2, o2))

    _zero_fill(zbuf, HDIM)
    _zero_acc(sid, zbuf, out_sh)

    for r, (xl_hbm, src_hbm, dst_hbm, ex_hbm, r_hbm, out_hbm) \
            in enumerate(rels):
        plsc.subcore_barrier()

        def blk_body(b, carry):
            pltpu.sync_copy(src_hbm.at[pl.ds(rb + b * BLK, BLK)], src_blk)
            pltpu.sync_copy(dst_hbm.at[pl.ds(rb + b * BLK, BLK)], dst_blk)
            lax.fori_loop(0, BLK, functools.partial(chunk, b), 0)
            return carry

        def chunk(b, j, carry):
            k = b * BLK + j
            cp1 = pltpu.async_copy(xl_hbm.at[src_blk.at[j]], xl_rows, sem1)
            cp2 = pltpu.async_copy(r_hbm.at[dst_blk.at[j]], r_rows, sem2)
            base_edge = (rb + k) * C
            pltpu.sync_copy(ex_hbm.at[pl.ds(base_edge, C)], ex_rows)
            cp1.wait()
            cp2.wait()
            for e in range(C):
                al_row = ex_rows[e, :] * r_rows[e, pl.ds(0, 16)]
                avs = [al_row[h] for h in range(HEADS)]
                for c4 in range(4):
                    acc = None
                    for h in range(HEADS):
                        x = xl_rows[e, pl.ds(h * HDIM + c4 * 16, 16)]
                        t = avs[h] * x
                        acc = t if acc is None else acc + t
                    vbuf[e, pl.ds(c4 * 16, 16)] = acc
            pltpu.sync_copy(vbuf, out_sh.at[dst_blk.at[j]], add=True)
            return carry

        lax.fori_loop(0, NBLK, blk_body, 0)

        plsc.subcore_barrier()
        _acc_slices(sid, lambda lo, sz: pltpu.sync_copy(
            out_sh.at[pl.ds(lo, sz)], out_hbm.at[cid, pl.ds(lo, sz)]))
        if r < 2:
            _zero_acc(sid, zbuf, out_sh)


def _sc_pass2(tables_l, edges2, exs, rs):
    o_t = jax.ShapeDtypeStruct((NCORE, N, HDIM), _f32)
    kern = pl.kernel(
        _pass2_body,
        out_type=(o_t, o_t, o_t),
        mesh=_sc_mesh(),
        compiler_params=pltpu.CompilerParams(needs_layout_passes=False),
        scratch_types=[
            pltpu.VMEM((BLK, C), _i32),
            pltpu.VMEM((BLK, C), _i32),
            pltpu.VMEM((C, FDIM), _f32),
            pltpu.VMEM((C, 16), _f32),
            pltpu.VMEM((C, 128), _f32),
            pltpu.VMEM((C, HDIM), _f32),
            pltpu.VMEM((16, HDIM), _f32),
            pltpu.VMEM_SHARED((N, HDIM), _f32),
            pltpu.SemaphoreType.DMA,
            pltpu.SemaphoreType.DMA,
        ],
    )
    (s0, d0), (s1, d1), (s2, d2) = edges2
    return kern(tables_l[0], tables_l[1], tables_l[2],
                s0, d0, s1, d1, s2, d2,
                exs[0], exs[1], exs[2], rs[0], rs[1], rs[2])


def _prep_edges(ei):
    src = jnp.asarray(ei[0], _i32)
    dst = jnp.asarray(ei[1], _i32)
    pad = E_PAD - E
    src = jnp.concatenate([src, jnp.zeros((pad,), _i32)])
    dst = jnp.concatenate([dst, jnp.zeros((pad,), _i32)])
    return src.reshape(E_PAD // C, C), dst.reshape(E_PAD // C, C)


def kernel(x_gene, x_protein, edge_index_gg, edge_index_gp, edge_index_pp,
           params):
    p = params
    edges2 = [_prep_edges(e)
              for e in (edge_index_gg, edge_index_gp, edge_index_pp)]
    zb = jnp.zeros((FDIM,), _f32)

    xg = _mm(x_gene, p['Wp_gene'], p['bp_gene'], act=True)
    xp = _mm(x_protein, p['Wp_protein'], p['bp_protein'], act=True)

    def layer(lname, xg_in, xp_in, slW_g, slb_g, slW_p, slb_p):
        tables = []
        atts = []
        for rel, (xs, xd) in (('gg', (xg_in, xg_in)),
                              ('gp', (xg_in, xp_in)),
                              ('pp', (xp_in, xp_in))):
            pre = rel + lname
            xl = _mm(xs, p[pre + '_Wl'], zb)
            xr = _mm(xd, p[pre + '_Wr'], zb)
            tables.append((xl, xr))
            atts.append(p[pre + '_att'].reshape(FDIM))
        ex0, ex1, ex2, dn0, dn1, dn2 = _sc_pass1(tables, edges2, atts)
        rs = [_recip(dn) for dn in (dn0, dn1, dn2)]
        og, ogp, opp = _sc_pass2([t[0] for t in tables], edges2,
                                 (ex0, ex1, ex2), rs)
        x_g = _tc_call(_comb_g_body, jax.ShapeDtypeStruct((N, HDIM), _f32),
                       og[0], og[1], xg_in, slW_g,
                       p['gg' + lname + '_b'].reshape(1, HDIM),
                       slb_g.reshape(1, HDIM))
        x_p = _tc_call(_comb_p_body, jax.ShapeDtypeStruct((N, HDIM), _f32),
                       ogp[0], ogp[1], opp[0], opp[1], xp_in, slW_p,
                       p['gp' + lname + '_b'].reshape(1, HDIM),
                       p['pp' + lname + '_b'].reshape(1, HDIM),
                       slb_p.reshape(1, HDIM))
        return x_g, x_p

    x1_g, x1_p = layer('1', xg, xp,
                       p['sl1_Wg'], p['sl1_bg'], p['sl1_Wp'], p['sl1_bp'])
    x2_g, x2_p = layer('2', x1_g, x1_p,
                       p['sl2_Wg'], p['sl2_bg'], p['sl2_Wp'], p['sl2_bp'])

    wg = p['W_int'][:HDIM]
    wp = p['W_int'][HDIM:]
    return _tc_call(_final_body, jax.ShapeDtypeStruct((N, 10), _f32),
                    x2_g, x2_p, wg, wp, p['b_int'].reshape(1, 10))


# speedup vs baseline: 2.2931x; 2.2931x over previous
import functools

import jax
import jax.numpy as jnp
from jax import lax
from jax.experimental import pallas as pl
from jax.experimental.pallas import tpu as pltpu
from jax.experimental.pallas import tpu_sc as plsc

N = 10000
E = 200000
HEADS = 4
HDIM = 64
FDIM = HEADS * HDIM

NCORE = 2
NSUB = 16
NTILE = NCORE * NSUB
C = 16
CHUNKS = 392
BLK = 56
NBLK = CHUNKS // BLK
EPT = C * CHUNKS
E_PAD = EPT * NTILE
RPS = 624
RTAIL = N - NSUB * RPS

_f32 = jnp.float32
_i32 = jnp.int32


def _tc_call(body, out_shapes, *args):
    return pl.pallas_call(body, out_shape=out_shapes)(*args)


def _mm_body(x_ref, w_ref, b_ref, o_ref, *, act):
    y = jnp.dot(x_ref[...], w_ref[...], preferred_element_type=_f32)
    y = y + b_ref[...]
    if act:
        y = jnp.maximum(y, 0.0)
    o_ref[...] = y


def _mm(x, w, b, act=False):
    body = functools.partial(_mm_body, act=act)
    return _tc_call(body, jax.ShapeDtypeStruct((x.shape[0], w.shape[1]), _f32),
                    x, w, b.reshape(1, -1))


def _recip_body(d_ref, o_ref):
    o_ref[...] = 1.0 / (d_ref[0] + d_ref[1] + 1e-16)


def _recip(den):
    d = den.reshape(2, (N * 16) // 128, 128)
    r = _tc_call(_recip_body, jax.ShapeDtypeStruct(d.shape[1:], _f32), d)
    return jnp.pad(r.reshape(N, 16), ((0, 0), (0, 112)))


def _comb_g_body(o0_ref, o1_ref, x_ref, w_ref, br_ref, bs_ref, out_ref):
    a = 0.25 * (o0_ref[...] + o1_ref[...]) + br_ref[...]
    y = a + jnp.dot(x_ref[...], w_ref[...], preferred_element_type=_f32) + bs_ref[...]
    out_ref[...] = jnp.maximum(y, 0.0)


def _comb_p_body(g0_ref, g1_ref, p0_ref, p1_ref, x_ref, w_ref,
                 b1_ref, b2_ref, bs_ref, out_ref):
    a = (0.25 * (g0_ref[...] + g1_ref[...]) + b1_ref[...]
         + 0.25 * (p0_ref[...] + p1_ref[...]) + b2_ref[...])
    y = a + jnp.dot(x_ref[...], w_ref[...], preferred_element_type=_f32) + bs_ref[...]
    out_ref[...] = jnp.maximum(y, 0.0)


def _final_body(xg_ref, xp_ref, wg_ref, wp_ref, b_ref, o_ref):
    y = (jnp.dot(xg_ref[...], wg_ref[...], preferred_element_type=_f32)
         + jnp.dot(xp_ref[...], wp_ref[...], preferred_element_type=_f32)
         + b_ref[...])
    o_ref[...] = y


def _sc_mesh():
    return plsc.VectorSubcoreMesh(core_axis_name="c", subcore_axis_name="s")


def _acc_slices(sid, fn):
    fn(sid * RPS, RPS)

    @pl.when(sid == NSUB - 1)
    def _tail():
        fn(NSUB * RPS, RTAIL)


def _zero_fill(zbuf, w):
    zero16 = jnp.zeros((16,), _f32)
    for i in range(16):
        for j in range(w // 16):
            zbuf[i, pl.ds(j * 16, 16)] = zero16


def _zero_acc(sid, zbuf, acc_sh):
    def zslices(lo, sz):
        def b(j, c):
            pltpu.sync_copy(zbuf, acc_sh.at[pl.ds(lo + j * 16, 16)])
            return c
        lax.fori_loop(0, sz // 16, b, 0)
    _acc_slices(sid, zslices)


def _pass1_body(xl0, xr0, xl1, xr1, xl2, xr2,
                s0, d0, s1, d1, s2, d2, a0, a1, a2,
                ex0, ex1, ex2, dn0, dn1, dn2,
                src_blk, dst_blk, xla, xlb, xra, xrb, att_v, exb, zbuf,
                den_sh, sxl0, sxl1, sxr0, sxr1):
    cid = lax.axis_index("c")
    sid = lax.axis_index("s")
    wid = sid * NCORE + cid
    rb = wid * CHUNKS

    rels = ((xl0, xr0, s0, d0, a0, ex0, dn0),
            (xl1, xr1, s1, d1, a1, ex1, dn1),
            (xl2, xr2, s2, d2, a2, ex2, dn2))

    xl_rows = (xla, xlb)
    xr_rows = (xra, xrb)
    sxl = (sxl0, sxl1)
    sxr = (sxr0, sxr1)

    _zero_fill(zbuf, 16)
    _zero_acc(sid, zbuf, den_sh)

    lane = lax.iota(_i32, 16)
    lanemask = lane < HEADS

    for r, (xl_hbm, xr_hbm, src_hbm, dst_hbm, att_hbm, ex_out, den_out) \
            in enumerate(rels):
        pltpu.sync_copy(att_hbm, att_v)
        plsc.subcore_barrier()

        attv = [att_v[pl.ds(i * 16, 16)] for i in range(16)]

        def issue(j, par):
            pltpu.async_copy(xl_hbm.at[src_blk.at[j]], xl_rows[par],
                             sxl[par])
            pltpu.async_copy(xr_hbm.at[dst_blk.at[j]], xr_rows[par],
                             sxr[par])

        def chunk(b, j, par):
            k = b * BLK + j
            pltpu.make_async_copy(xl_hbm.at[src_blk.at[0]],
                                  xl_rows[par], sxl[par]).wait()
            pltpu.make_async_copy(xr_hbm.at[dst_blk.at[0]],
                                  xr_rows[par], sxr[par]).wait()
            base_edge = (rb + k) * C
            for e in range(C):
                row = jnp.zeros((16,), _f32)
                for h in range(HEADS):
                    acc = None
                    for c4 in range(4):
                        off = h * HDIM + c4 * 16
                        s = (xl_rows[par][e, pl.ds(off, 16)]
                             + xr_rows[par][e, pl.ds(off, 16)])
                        s = jnp.maximum(s, s * 0.2)
                        t = s * attv[h * 4 + c4]
                        acc = t if acc is None else acc + t
                    row = jnp.where(lane == h, jnp.sum(acc), row)
                ex = jnp.exp(row)
                m = jnp.logical_and(lanemask, base_edge + e < E)
                exb[e, :] = jnp.where(m, ex, 0.0)
            pltpu.sync_copy(exb, ex_out.at[pl.ds(base_edge, C)])
            pltpu.sync_copy(exb, den_sh.at[dst_blk.at[j]], add=True)

            @pl.when(j + 2 < BLK)
            def _prefetch():
                issue(j + 2, par)

        def blk_body(b, carry):
            pltpu.sync_copy(src_hbm.at[pl.ds(rb + b * BLK, BLK)], src_blk)
            pltpu.sync_copy(dst_hbm.at[pl.ds(rb + b * BLK, BLK)], dst_blk)
            for par in (0, 1):
                issue(par, par)

            def pair(jj, c2):
                for par in (0, 1):
                    chunk(b, 2 * jj + par, par)
                return c2

            lax.fori_loop(0, BLK // 2, pair, 0)
            return carry

        lax.fori_loop(0, NBLK, blk_body, 0)

        plsc.subcore_barrier()
        _acc_slices(sid, lambda lo, sz: pltpu.sync_copy(
            den_sh.at[pl.ds(lo, sz)], den_out.at[cid, pl.ds(lo, sz)]))
        if r < 2:
            _zero_acc(sid, zbuf, den_sh)


def _sc_pass1(tables, edges2, atts):
    ex_t = jax.ShapeDtypeStruct((E_PAD, 16), _f32)
    dn_t = jax.ShapeDtypeStruct((NCORE, N, 16), _f32)
    kern = pl.kernel(
        _pass1_body,
        out_type=(ex_t, ex_t, ex_t, dn_t, dn_t, dn_t),
        mesh=_sc_mesh(),
        compiler_params=pltpu.CompilerParams(needs_layout_passes=False),
        scratch_types=[
            pltpu.VMEM((BLK, C), _i32),
            pltpu.VMEM((BLK, C), _i32),
            pltpu.VMEM((C, FDIM), _f32),
            pltpu.VMEM((C, FDIM), _f32),
            pltpu.VMEM((C, FDIM), _f32),
            pltpu.VMEM((C, FDIM), _f32),
            pltpu.VMEM((FDIM,), _f32),
            pltpu.VMEM((C, 16), _f32),
            pltpu.VMEM((16, 16), _f32),
            pltpu.VMEM_SHARED((N, 16), _f32),
            pltpu.SemaphoreType.DMA,
            pltpu.SemaphoreType.DMA,
            pltpu.SemaphoreType.DMA,
            pltpu.SemaphoreType.DMA,
        ],
    )
    (xl0, xr0), (xl1, xr1), (xl2, xr2) = tables
    (s0, d0), (s1, d1), (s2, d2) = edges2
    return kern(xl0, xr0, xl1, xr1, xl2, xr2,
                s0, d0, s1, d1, s2, d2, atts[0], atts[1], atts[2])


def _pass2_body(xl0, xl1, xl2, s0, d0, s1, d1, s2, d2,
                e0, e1, e2, r0, r1, r2,
                o0, o1, o2,
                src_blk, dst_blk, xla, xlb, exa, exbb, ra, rbb, vbuf, zbuf,
                out_sh, sxl0, sxl1, se0, se1, sr0, sr1):
    cid = lax.axis_index("c")
    sid = lax.axis_index("s")
    wid = sid * NCORE + cid
    rb = wid * CHUNKS

    rels = ((xl0, s0, d0, e0, r0, o0),
            (xl1, s1, d1, e1, r1, o1),
            (xl2, s2, d2, e2, r2, o2))

    xl_rows = (xla, xlb)
    ex_rows = (exa, exbb)
    r_rows = (ra, rbb)
    sxl = (sxl0, sxl1)
    se = (se0, se1)
    sr = (sr0, sr1)

    _zero_fill(zbuf, HDIM)
    _zero_acc(sid, zbuf, out_sh)

    for r, (xl_hbm, src_hbm, dst_hbm, ex_hbm, r_hbm, out_hbm) \
            in enumerate(rels):
        plsc.subcore_barrier()

        def issue(b, j, par):
            pltpu.async_copy(xl_hbm.at[src_blk.at[j]], xl_rows[par],
                             sxl[par])
            pltpu.async_copy(r_hbm.at[dst_blk.at[j]], r_rows[par], sr[par])
            pltpu.async_copy(ex_hbm.at[pl.ds((rb + b * BLK + j) * C, C)],
                             ex_rows[par], se[par])

        def chunk(b, j, par):
            pltpu.make_async_copy(xl_hbm.at[src_blk.at[0]],
                                  xl_rows[par], sxl[par]).wait()
            pltpu.make_async_copy(r_hbm.at[dst_blk.at[0]],
                                  r_rows[par], sr[par]).wait()
            pltpu.make_async_copy(ex_hbm.at[pl.ds(0, C)],
                                  ex_rows[par], se[par]).wait()
            for e in range(C):
                al_row = ex_rows[par][e, :] * r_rows[par][e, pl.ds(0, 16)]
                avs = [al_row[h] for h in range(HEADS)]
                for c4 in range(4):
                    acc = None
                    for h in range(HEADS):
                        x = xl_rows[par][e, pl.ds(h * HDIM + c4 * 16, 16)]
                        t = avs[h] * x
                        acc = t if acc is None else acc + t
                    vbuf[e, pl.ds(c4 * 16, 16)] = acc
            pltpu.sync_copy(vbuf, out_sh.at[dst_blk.at[j]], add=True)

            @pl.when(j + 2 < BLK)
            def _prefetch():
                issue(b, j + 2, par)

        def blk_body(b, carry):
            pltpu.sync_copy(src_hbm.at[pl.ds(rb + b * BLK, BLK)], src_blk)
            pltpu.sync_copy(dst_hbm.at[pl.ds(rb + b * BLK, BLK)], dst_blk)
            for par in (0, 1):
                issue(b, par, par)

            def pair(jj, c2):
                for par in (0, 1):
                    chunk(b, 2 * jj + par, par)
                return c2

            lax.fori_loop(0, BLK // 2, pair, 0)
            return carry

        lax.fori_loop(0, NBLK, blk_body, 0)

        plsc.subcore_barrier()
        _acc_slices(sid, lambda lo, sz: pltpu.sync_copy(
            out_sh.at[pl.ds(lo, sz)], out_hbm.at[cid, pl.ds(lo, sz)]))
        if r < 2:
            _zero_acc(sid, zbuf, out_sh)


def _sc_pass2(tables_l, edges2, exs, rs):
    o_t = jax.ShapeDtypeStruct((NCORE, N, HDIM), _f32)
    kern = pl.kernel(
        _pass2_body,
        out_type=(o_t, o_t, o_t),
        mesh=_sc_mesh(),
        compiler_params=pltpu.CompilerParams(needs_layout_passes=False),
        scratch_types=[
            pltpu.VMEM((BLK, C), _i32),
            pltpu.VMEM((BLK, C), _i32),
            pltpu.VMEM((C, FDIM), _f32),
            pltpu.VMEM((C, FDIM), _f32),
            pltpu.VMEM((C, 16), _f32),
            pltpu.VMEM((C, 16), _f32),
            pltpu.VMEM((C, 128), _f32),
            pltpu.VMEM((C, 128), _f32),
            pltpu.VMEM((C, HDIM), _f32),
            pltpu.VMEM((16, HDIM), _f32),
            pltpu.VMEM_SHARED((N, HDIM), _f32),
            pltpu.SemaphoreType.DMA,
            pltpu.SemaphoreType.DMA,
            pltpu.SemaphoreType.DMA,
            pltpu.SemaphoreType.DMA,
            pltpu.SemaphoreType.DMA,
            pltpu.SemaphoreType.DMA,
        ],
    )
    (s0, d0), (s1, d1), (s2, d2) = edges2
    return kern(tables_l[0], tables_l[1], tables_l[2],
                s0, d0, s1, d1, s2, d2,
                exs[0], exs[1], exs[2], rs[0], rs[1], rs[2])


def _prep_edges(ei):
    src = jnp.asarray(ei[0], _i32)
    dst = jnp.asarray(ei[1], _i32)
    pad = E_PAD - E
    src = jnp.concatenate([src, jnp.zeros((pad,), _i32)])
    dst = jnp.concatenate([dst, jnp.zeros((pad,), _i32)])
    return src.reshape(E_PAD // C, C), dst.reshape(E_PAD // C, C)


def kernel(x_gene, x_protein, edge_index_gg, edge_index_gp, edge_index_pp,
           params):
    p = params
    edges2 = [_prep_edges(e)
              for e in (edge_index_gg, edge_index_gp, edge_index_pp)]
    zb = jnp.zeros((FDIM,), _f32)

    xg = _mm(x_gene, p['Wp_gene'], p['bp_gene'], act=True)
    xp = _mm(x_protein, p['Wp_protein'], p['bp_protein'], act=True)

    def layer(lname, xg_in, xp_in, slW_g, slb_g, slW_p, slb_p):
        tables = []
        atts = []
        for rel, (xs, xd) in (('gg', (xg_in, xg_in)),
                              ('gp', (xg_in, xp_in)),
                              ('pp', (xp_in, xp_in))):
            pre = rel + lname
            xl = _mm(xs, p[pre + '_Wl'], zb)
            xr = _mm(xd, p[pre + '_Wr'], zb)
            tables.append((xl, xr))
            atts.append(p[pre + '_att'].reshape(FDIM))
        ex0, ex1, ex2, dn0, dn1, dn2 = _sc_pass1(tables, edges2, atts)
        rs = [_recip(dn) for dn in (dn0, dn1, dn2)]
        og, ogp, opp = _sc_pass2([t[0] for t in tables], edges2,
                                 (ex0, ex1, ex2), rs)
        x_g = _tc_call(_comb_g_body, jax.ShapeDtypeStruct((N, HDIM), _f32),
                       og[0], og[1], xg_in, slW_g,
                       p['gg' + lname + '_b'].reshape(1, HDIM),
                       slb_g.reshape(1, HDIM))
        x_p = _tc_call(_comb_p_body, jax.ShapeDtypeStruct((N, HDIM), _f32),
                       ogp[0], ogp[1], opp[0], opp[1], xp_in, slW_p,
                       p['gp' + lname + '_b'].reshape(1, HDIM),
                       p['pp' + lname + '_b'].reshape(1, HDIM),
                       slb_p.reshape(1, HDIM))
        return x_g, x_p

    x1_g, x1_p = layer('1', xg, xp,
                       p['sl1_Wg'], p['sl1_bg'], p['sl1_Wp'], p['sl1_bp'])
    x2_g, x2_p = layer('2', x1_g, x1_p,
                       p['sl2_Wg'], p['sl2_bg'], p['sl2_Wp'], p['sl2_bp'])

    wg = p['W_int'][:HDIM]
    wp = p['W_int'][HDIM:]
    return _tc_call(_final_body, jax.ShapeDtypeStruct((N, 10), _f32),
                    x2_g, x2_p, wg, wp, p['b_int'].reshape(1, 10))


# speedup vs baseline: 2.3590x; 1.0287x over previous
import functools

import jax
import jax.numpy as jnp
from jax import lax
from jax.experimental import pallas as pl
from jax.experimental.pallas import tpu as pltpu
from jax.experimental.pallas import tpu_sc as plsc

N = 10000
E = 200000
HEADS = 4
HDIM = 64
FDIM = HEADS * HDIM

NCORE = 2
NSUB = 16
NTILE = NCORE * NSUB
C = 16
CHUNKS = 392
BLK = 56
NBLK = CHUNKS // BLK
EPT = C * CHUNKS
E_PAD = EPT * NTILE
RPS = 624
RTAIL = N - NSUB * RPS

_f32 = jnp.float32
_i32 = jnp.int32


def _tc_call(body, out_shapes, *args):
    return pl.pallas_call(body, out_shape=out_shapes)(*args)


def _mm_body(x_ref, w_ref, b_ref, o_ref, *, act):
    y = jnp.dot(x_ref[...], w_ref[...], preferred_element_type=_f32)
    y = y + b_ref[...]
    if act:
        y = jnp.maximum(y, 0.0)
    o_ref[...] = y


def _mm(x, w, b, act=False):
    body = functools.partial(_mm_body, act=act)
    return _tc_call(body, jax.ShapeDtypeStruct((x.shape[0], w.shape[1]), _f32),
                    x, w, b.reshape(1, -1))


def _recip_body(d_ref, o_ref):
    o_ref[...] = 1.0 / (d_ref[0] + d_ref[1] + 1e-16)


def _recip(den):
    d = den.reshape(2, (N * 16) // 128, 128)
    r = _tc_call(_recip_body, jax.ShapeDtypeStruct(d.shape[1:], _f32), d)
    return jnp.pad(r.reshape(N, 16), ((0, 0), (0, 112)))


def _comb_g_body(o0_ref, o1_ref, x_ref, w_ref, br_ref, bs_ref, out_ref):
    a = 0.25 * (o0_ref[...] + o1_ref[...]) + br_ref[...]
    y = a + jnp.dot(x_ref[...], w_ref[...], preferred_element_type=_f32) + bs_ref[...]
    out_ref[...] = jnp.maximum(y, 0.0)


def _comb_p_body(g0_ref, g1_ref, p0_ref, p1_ref, x_ref, w_ref,
                 b1_ref, b2_ref, bs_ref, out_ref):
    a = (0.25 * (g0_ref[...] + g1_ref[...]) + b1_ref[...]
         + 0.25 * (p0_ref[...] + p1_ref[...]) + b2_ref[...])
    y = a + jnp.dot(x_ref[...], w_ref[...], preferred_element_type=_f32) + bs_ref[...]
    out_ref[...] = jnp.maximum(y, 0.0)


def _final_body(xg_ref, xp_ref, wg_ref, wp_ref, b_ref, o_ref):
    y = (jnp.dot(xg_ref[...], wg_ref[...], preferred_element_type=_f32)
         + jnp.dot(xp_ref[...], wp_ref[...], preferred_element_type=_f32)
         + b_ref[...])
    o_ref[...] = y


def _sc_mesh():
    return plsc.VectorSubcoreMesh(core_axis_name="c", subcore_axis_name="s")


def _acc_slices(sid, fn):
    fn(sid * RPS, RPS)

    @pl.when(sid == NSUB - 1)
    def _tail():
        fn(NSUB * RPS, RTAIL)


def _zero_fill(zbuf, w):
    zero16 = jnp.zeros((16,), _f32)
    for i in range(16):
        for j in range(w // 16):
            zbuf[i, pl.ds(j * 16, 16)] = zero16


def _zero_acc(sid, zbuf, acc_sh):
    def zslices(lo, sz):
        def b(j, c):
            pltpu.sync_copy(zbuf, acc_sh.at[pl.ds(lo + j * 16, 16)])
            return c
        lax.fori_loop(0, sz // 16, b, 0)
    _acc_slices(sid, zslices)


def _pass1_body(xl0, xr0, xl1, xr1, xl2, xr2,
                s0, d0, s1, d1, s2, d2, a0, a1, a2,
                ex0, ex1, ex2, dn0, dn1, dn2,
                src_blk, dst_blk, xla, xlb, xra, xrb, att_v,
                exba, exbb, zbuf,
                den_sh, sxl0, sxl1, sxr0, sxr1, sex0, sex1):
    cid = lax.axis_index("c")
    sid = lax.axis_index("s")
    wid = sid * NCORE + cid
    rb = wid * CHUNKS

    rels = ((xl0, xr0, s0, d0, a0, ex0, dn0),
            (xl1, xr1, s1, d1, a1, ex1, dn1),
            (xl2, xr2, s2, d2, a2, ex2, dn2))

    xl_rows = (xla, xlb)
    xr_rows = (xra, xrb)
    exbs = (exba, exbb)
    sxl = (sxl0, sxl1)
    sxr = (sxr0, sxr1)
    sex = (sex0, sex1)

    _zero_fill(zbuf, 16)
    _zero_acc(sid, zbuf, den_sh)

    lane = lax.iota(_i32, 16)
    lanemask = lane < HEADS

    for r, (xl_hbm, xr_hbm, src_hbm, dst_hbm, att_hbm, ex_out, den_out) \
            in enumerate(rels):
        pltpu.sync_copy(att_hbm, att_v)
        plsc.subcore_barrier()

        attv = [att_v[pl.ds(i * 16, 16)] for i in range(16)]

        def issue(j, par):
            pltpu.async_copy(xl_hbm.at[src_blk.at[j]], xl_rows[par],
                             sxl[par])
            pltpu.async_copy(xr_hbm.at[dst_blk.at[j]], xr_rows[par],
                             sxr[par])

        def chunk(b, j, par):
            k = b * BLK + j
            pltpu.make_async_copy(xl_hbm.at[src_blk.at[0]],
                                  xl_rows[par], sxl[par]).wait()
            pltpu.make_async_copy(xr_hbm.at[dst_blk.at[0]],
                                  xr_rows[par], sxr[par]).wait()

            @pl.when(k >= 2)
            def _wex():
                pltpu.make_async_copy(exbs[par], ex_out.at[pl.ds(0, C)],
                                      sex[par]).wait()

            base_edge = (rb + k) * C
            for e in range(C):
                row = jnp.zeros((16,), _f32)
                for h in range(HEADS):
                    acc = None
                    for c4 in range(4):
                        off = h * HDIM + c4 * 16
                        s = (xl_rows[par][e, pl.ds(off, 16)]
                             + xr_rows[par][e, pl.ds(off, 16)])
                        s = jnp.maximum(s, s * 0.2)
                        t = s * attv[h * 4 + c4]
                        acc = t if acc is None else acc + t
                    row = jnp.where(lane == h, jnp.sum(acc), row)
                ex = jnp.exp(row)
                m = jnp.logical_and(lanemask, base_edge + e < E)
                exbs[par][e, :] = jnp.where(m, ex, 0.0)
            pltpu.sync_copy(exbs[par], den_sh.at[dst_blk.at[j]], add=True)
            pltpu.async_copy(exbs[par], ex_out.at[pl.ds(base_edge, C)],
                             sex[par])

            @pl.when(j + 2 < BLK)
            def _prefetch():
                issue(j + 2, par)

        def blk_body(b, carry):
            pltpu.sync_copy(src_hbm.at[pl.ds(rb + b * BLK, BLK)], src_blk)
            pltpu.sync_copy(dst_hbm.at[pl.ds(rb + b * BLK, BLK)], dst_blk)
            for par in (0, 1):
                issue(par, par)

            def pair(jj, c2):
                for par in (0, 1):
                    chunk(b, 2 * jj + par, par)
                return c2

            lax.fori_loop(0, BLK // 2, pair, 0)
            return carry

        lax.fori_loop(0, NBLK, blk_body, 0)

        for par in (0, 1):
            pltpu.make_async_copy(exbs[par], ex_out.at[pl.ds(0, C)],
                                  sex[par]).wait()

        plsc.subcore_barrier()
        _acc_slices(sid, lambda lo, sz: pltpu.sync_copy(
            den_sh.at[pl.ds(lo, sz)], den_out.at[cid, pl.ds(lo, sz)]))
        if r < 2:
            _zero_acc(sid, zbuf, den_sh)


def _sc_pass1(tables, edges2, atts):
    ex_t = jax.ShapeDtypeStruct((E_PAD, 16), _f32)
    dn_t = jax.ShapeDtypeStruct((NCORE, N, 16), _f32)
    kern = pl.kernel(
        _pass1_body,
        out_type=(ex_t, ex_t, ex_t, dn_t, dn_t, dn_t),
        mesh=_sc_mesh(),
        compiler_params=pltpu.CompilerParams(needs_layout_passes=False),
        scratch_types=[
            pltpu.VMEM((BLK, C), _i32),
            pltpu.VMEM((BLK, C), _i32),
            pltpu.VMEM((C, FDIM), _f32),
            pltpu.VMEM((C, FDIM), _f32),
            pltpu.VMEM((C, FDIM), _f32),
            pltpu.VMEM((C, FDIM), _f32),
            pltpu.VMEM((FDIM,), _f32),
            pltpu.VMEM((C, 16), _f32),
            pltpu.VMEM((C, 16), _f32),
            pltpu.VMEM((16, 16), _f32),
            pltpu.VMEM_SHARED((N, 16), _f32),
            pltpu.SemaphoreType.DMA,
            pltpu.SemaphoreType.DMA,
            pltpu.SemaphoreType.DMA,
            pltpu.SemaphoreType.DMA,
            pltpu.SemaphoreType.DMA,
            pltpu.SemaphoreType.DMA,
        ],
    )
    (xl0, xr0), (xl1, xr1), (xl2, xr2) = tables
    (s0, d0), (s1, d1), (s2, d2) = edges2
    return kern(xl0, xr0, xl1, xr1, xl2, xr2,
                s0, d0, s1, d1, s2, d2, atts[0], atts[1], atts[2])


def _pass2_body(xl0, xl1, xl2, s0, d0, s1, d1, s2, d2,
                e0, e1, e2, r0, r1, r2,
                o0, o1, o2,
                src_blk, dst_blk, xla, xlb, exa, exbb, ra, rbb, vbuf, zbuf,
                out_sh, sxl0, sxl1, se0, se1, sr0, sr1):
    cid = lax.axis_index("c")
    sid = lax.axis_index("s")
    wid = sid * NCORE + cid
    rb = wid * CHUNKS

    rels = ((xl0, s0, d0, e0, r0, o0),
            (xl1, s1, d1, e1, r1, o1),
            (xl2, s2, d2, e2, r2, o2))

    xl_rows = (xla, xlb)
    ex_rows = (exa, exbb)
    r_rows = (ra, rbb)
    sxl = (sxl0, sxl1)
    se = (se0, se1)
    sr = (sr0, sr1)

    _zero_fill(zbuf, HDIM)
    _zero_acc(sid, zbuf, out_sh)

    for r, (xl_hbm, src_hbm, dst_hbm, ex_hbm, r_hbm, out_hbm) \
            in enumerate(rels):
        plsc.subcore_barrier()

        def issue(b, j, par):
            pltpu.async_copy(xl_hbm.at[src_blk.at[j]], xl_rows[par],
                             sxl[par])
            pltpu.async_copy(r_hbm.at[dst_blk.at[j]], r_rows[par], sr[par])
            pltpu.async_copy(ex_hbm.at[pl.ds((rb + b * BLK + j) * C, C)],
                             ex_rows[par], se[par])

        def chunk(b, j, par):
            pltpu.make_async_copy(xl_hbm.at[src_blk.at[0]],
                                  xl_rows[par], sxl[par]).wait()
            pltpu.make_async_copy(r_hbm.at[dst_blk.at[0]],
                                  r_rows[par], sr[par]).wait()
            pltpu.make_async_copy(ex_hbm.at[pl.ds(0, C)],
                                  ex_rows[par], se[par]).wait()
            for e in range(C):
                al_row = ex_rows[par][e, :] * r_rows[par][e, pl.ds(0, 16)]
                avs = [al_row[h] for h in range(HEADS)]
                for c4 in range(4):
                    acc = None
                    for h in range(HEADS):
                        x = xl_rows[par][e, pl.ds(h * HDIM + c4 * 16, 16)]
                        t = avs[h] * x
                        acc = t if acc is None else acc + t
                    vbuf[e, pl.ds(c4 * 16, 16)] = acc
            pltpu.sync_copy(vbuf, out_sh.at[dst_blk.at[j]], add=True)

            @pl.when(j + 2 < BLK)
            def _prefetch():
                issue(b, j + 2, par)

        def blk_body(b, carry):
            pltpu.sync_copy(src_hbm.at[pl.ds(rb + b * BLK, BLK)], src_blk)
            pltpu.sync_copy(dst_hbm.at[pl.ds(rb + b * BLK, BLK)], dst_blk)
            for par in (0, 1):
                issue(b, par, par)

            def pair(jj, c2):
                for par in (0, 1):
                    chunk(b, 2 * jj + par, par)
                return c2

            lax.fori_loop(0, BLK // 2, pair, 0)
            return carry

        lax.fori_loop(0, NBLK, blk_body, 0)

        plsc.subcore_barrier()
        _acc_slices(sid, lambda lo, sz: pltpu.sync_copy(
            out_sh.at[pl.ds(lo, sz)], out_hbm.at[cid, pl.ds(lo, sz)]))
        if r < 2:
            _zero_acc(sid, zbuf, out_sh)


def _sc_pass2(tables_l, edges2, exs, rs):
    o_t = jax.ShapeDtypeStruct((NCORE, N, HDIM), _f32)
    kern = pl.kernel(
        _pass2_body,
        out_type=(o_t, o_t, o_t),
        mesh=_sc_mesh(),
        compiler_params=pltpu.CompilerParams(needs_layout_passes=False),
        scratch_types=[
            pltpu.VMEM((BLK, C), _i32),
            pltpu.VMEM((BLK, C), _i32),
            pltpu.VMEM((C, FDIM), _f32),
            pltpu.VMEM((C, FDIM), _f32),
            pltpu.VMEM((C, 16), _f32),
            pltpu.VMEM((C, 16), _f32),
            pltpu.VMEM((C, 128), _f32),
            pltpu.VMEM((C, 128), _f32),
            pltpu.VMEM((C, HDIM), _f32),
            pltpu.VMEM((16, HDIM), _f32),
            pltpu.VMEM_SHARED((N, HDIM), _f32),
            pltpu.SemaphoreType.DMA,
            pltpu.SemaphoreType.DMA,
            pltpu.SemaphoreType.DMA,
            pltpu.SemaphoreType.DMA,
            pltpu.SemaphoreType.DMA,
            pltpu.SemaphoreType.DMA,
        ],
    )
    (s0, d0), (s1, d1), (s2, d2) = edges2
    return kern(tables_l[0], tables_l[1], tables_l[2],
                s0, d0, s1, d1, s2, d2,
                exs[0], exs[1], exs[2], rs[0], rs[1], rs[2])


def _prep_edges(ei):
    src = jnp.asarray(ei[0], _i32)
    dst = jnp.asarray(ei[1], _i32)
    pad = E_PAD - E
    src = jnp.concatenate([src, jnp.zeros((pad,), _i32)])
    dst = jnp.concatenate([dst, jnp.zeros((pad,), _i32)])
    return src.reshape(E_PAD // C, C), dst.reshape(E_PAD // C, C)


def kernel(x_gene, x_protein, edge_index_gg, edge_index_gp, edge_index_pp,
           params):
    p = params
    edges2 = [_prep_edges(e)
              for e in (edge_index_gg, edge_index_gp, edge_index_pp)]
    zb = jnp.zeros((FDIM,), _f32)

    xg = _mm(x_gene, p['Wp_gene'], p['bp_gene'], act=True)
    xp = _mm(x_protein, p['Wp_protein'], p['bp_protein'], act=True)

    def layer(lname, xg_in, xp_in, slW_g, slb_g, slW_p, slb_p):
        tables = []
        atts = []
        for rel, (xs, xd) in (('gg', (xg_in, xg_in)),
                              ('gp', (xg_in, xp_in)),
                              ('pp', (xp_in, xp_in))):
            pre = rel + lname
            xl = _mm(xs, p[pre + '_Wl'], zb)
            xr = _mm(xd, p[pre + '_Wr'], zb)
            tables.append((xl, xr))
            atts.append(p[pre + '_att'].reshape(FDIM))
        ex0, ex1, ex2, dn0, dn1, dn2 = _sc_pass1(tables, edges2, atts)
        rs = [_recip(dn) for dn in (dn0, dn1, dn2)]
        og, ogp, opp = _sc_pass2([t[0] for t in tables], edges2,
                                 (ex0, ex1, ex2), rs)
        x_g = _tc_call(_comb_g_body, jax.ShapeDtypeStruct((N, HDIM), _f32),
                       og[0], og[1], xg_in, slW_g,
                       p['gg' + lname + '_b'].reshape(1, HDIM),
                       slb_g.reshape(1, HDIM))
        x_p = _tc_call(_comb_p_body, jax.ShapeDtypeStruct((N, HDIM), _f32),
                       ogp[0], ogp[1], opp[0], opp[1], xp_in, slW_p,
                       p['gp' + lname + '_b'].reshape(1, HDIM),
                       p['pp' + lname + '_b'].reshape(1, HDIM),
                       slb_p.reshape(1, HDIM))
        return x_g, x_p

    x1_g, x1_p = layer('1', xg, xp,
                       p['sl1_Wg'], p['sl1_bg'], p['sl1_Wp'], p['sl1_bp'])
    x2_g, x2_p = layer('2', x1_g, x1_p,
                       p['sl2_Wg'], p['sl2_bg'], p['sl2_Wp'], p['sl2_bp'])

    wg = p['W_int'][:HDIM]
    wp = p['W_int'][HDIM:]
    return _tc_call(_final_body, jax.ShapeDtypeStruct((N, 10), _f32),
                    x2_g, x2_p, wg, wp, p['b_int'].reshape(1, 10))
